# baseline (speedup 1.0000x reference)
import functools

import jax
import jax.numpy as jnp
from jax import lax
from jax.experimental import pallas as pl

_G = 4
_DIV = -0.5
_NTOP = 8
_BIG = 2**30
_NEG_INF = float("-inf")


def _topk_body(lp_ref, mask_ref, topv_ref, topi_ref, *, n_top, nseg, segw):
    x = lp_ref[0]
    mb = mask_ref[0, 0, :]
    nbeams = x.shape[0]
    x = jnp.where(mb[:, None, None] > 0, x, 0.0)
    m_seg = jnp.max(x, axis=2)
    x_flat = x.reshape(nbeams * nseg, segw)

    s_iota = lax.broadcasted_iota(jnp.int32, (nbeams, nseg), 1)
    w_iota = lax.broadcasted_iota(jnp.int32, (nbeams, segw), 1)
    c_iota = lax.broadcasted_iota(jnp.int32, (nbeams, nbeams * nseg), 1)
    row_base = lax.broadcasted_iota(jnp.int32, (nbeams,), 0) * nseg

    vals, idxs, picked = [], [], []
    for _ in range(n_top):
        m = jnp.max(m_seg, axis=1)
        s_star = jnp.min(
            jnp.where(m_seg == m[:, None], s_iota, _BIG), axis=1)
        oh = jnp.where(c_iota == (row_base + s_star)[:, None], 1.0, 0.0)
        y = lax.dot_general(oh, x_flat, (((1,), (0,)), ((), ())),
                            precision=lax.Precision.HIGHEST,
                            preferred_element_type=jnp.float32)
        flat_y = s_star[:, None] * segw + w_iota
        for p in picked:
            y = jnp.where(flat_y == p[:, None], _NEG_INF, y)
        w_star = jnp.min(
            jnp.where(y == m[:, None], w_iota, _BIG), axis=1)
        flat = s_star * segw + w_star
        vals.append(m)
        idxs.append(flat)
        picked.append(flat)
        y = jnp.where(w_iota == w_star[:, None], _NEG_INF, y)
        m_new = jnp.max(y, axis=1)
        m_seg = jnp.where(s_iota == s_star[:, None], m_new[:, None], m_seg)
    topv_ref[0] = jnp.stack(vals, axis=1)
    topi_ref[0] = jnp.stack(idxs, axis=1)


def _merge_body(topv_ref, topi_ref, scores_ref, sc_ref, ix_ref, bm_ref, *, vocab):
    tv = topv_ref[...]
    ti = topi_ref[...]
    s = scores_ref[...]
    bsz, _, n_top = tv.shape
    prev = []
    cols = {}
    for g in range(_G):
        b0, b1 = g, g + _G
        cv = jnp.concatenate(
            [tv[:, b0, :] + s[:, b0][:, None], tv[:, b1, :] + s[:, b1][:, None]],
            axis=1)
        ci = jnp.concatenate([ti[:, b0, :], ti[:, b1, :]], axis=1)
        cflat = jnp.concatenate([ti[:, b0, :], ti[:, b1, :] + vocab], axis=1)
        cbeam = jnp.concatenate(
            [jnp.full((bsz, n_top), b0, jnp.int32),
             jnp.full((bsz, n_top), b1, jnp.int32)], axis=1)
        pen = jnp.zeros(cv.shape, jnp.float32)
        for p in prev:
            pen = pen + jnp.where(ci == p[:, None], 1.0, 0.0)
        adj = cv + _DIV * pen
        picked = []
        for r in range(2):
            m1 = jnp.max(adj, axis=1)
            sel = adj == m1[:, None]
            f1 = jnp.min(jnp.where(sel, cflat, _BIG), axis=1)
            ch = cflat == f1[:, None]
            isel = jnp.sum(jnp.where(ch, ci, 0), axis=1)
            bsel = jnp.sum(jnp.where(ch, cbeam, 0), axis=1)
            cols[r * _G + g] = (m1, isel, bsel)
            adj = jnp.where(ch, _NEG_INF, adj)
            picked.append(isel)
        prev.extend(picked)
    sc_ref[...] = jnp.stack([cols[c][0] for c in range(2 * _G)], axis=1)
    ix_ref[...] = jnp.stack([cols[c][1] for c in range(2 * _G)], axis=1)
    bm_ref[...] = jnp.stack([cols[c][2] for c in range(2 * _G)], axis=1)


def kernel(step, lprobs, scores, mask):
    bsz, nbeams, vocab = lprobs.shape
    scores2 = jnp.squeeze(scores, -1).astype(jnp.float32)
    mask_f = mask.astype(jnp.float32).reshape(bsz, 1, nbeams)

    nseg, segw = 100, vocab // 100
    lp4 = lprobs.reshape(bsz, nbeams, nseg, segw)
    topv, topi = pl.pallas_call(
        functools.partial(_topk_body, n_top=_NTOP, nseg=nseg, segw=segw),
        grid=(bsz,),
        in_specs=[
            pl.BlockSpec((1, nbeams, nseg, segw), lambda b: (b, 0, 0, 0)),
            pl.BlockSpec((1, 1, nbeams), lambda b: (b, 0, 0)),
        ],
        out_specs=[
            pl.BlockSpec((1, nbeams, _NTOP), lambda b: (b, 0, 0)),
            pl.BlockSpec((1, nbeams, _NTOP), lambda b: (b, 0, 0)),
        ],
        out_shape=[
            jax.ShapeDtypeStruct((bsz, nbeams, _NTOP), jnp.float32),
            jax.ShapeDtypeStruct((bsz, nbeams, _NTOP), jnp.int32),
        ],
    )(lp4, mask_f)

    sc_out, ix_out, bm_out = pl.pallas_call(
        functools.partial(_merge_body, vocab=vocab),
        out_shape=[
            jax.ShapeDtypeStruct((bsz, 2 * _G), jnp.float32),
            jax.ShapeDtypeStruct((bsz, 2 * _G), jnp.int32),
            jax.ShapeDtypeStruct((bsz, 2 * _G), jnp.int32),
        ],
    )(topv, topi, scores2)
    return (sc_out, ix_out, bm_out)


# speedup vs baseline: 1.9093x; 1.9093x over previous
import functools

import jax
import jax.numpy as jnp
from jax import lax
from jax.experimental import pallas as pl

_G = 4
_DIV = -0.5
_NTOP = 8
_BIG = 2**30
_NEG_INF = float("-inf")


def _topk_body(lp_ref, mask_ref, topv_ref, topi_ref, *, n_top):
    x = lp_ref[0]
    mb = mask_ref[0, 0, :]
    x = jnp.where(mb[:, None] > 0, x, 0.0)
    v = x.shape[1]
    iota = lax.broadcasted_iota(jnp.int32, x.shape, 1)
    step_ = ((v // 4) // 128) * 128
    bounds = [0, step_, 2 * step_, 3 * step_, v]
    vals, idxs = [], []
    for _ in range(n_top):
        pm = [jnp.max(x[:, a:b], axis=1) for a, b in zip(bounds, bounds[1:])]
        m = jnp.maximum(jnp.maximum(pm[0], pm[1]), jnp.maximum(pm[2], pm[3]))
        pc = [
            jnp.min(jnp.where(x[:, a:b] == m[:, None], iota[:, a:b], _BIG),
                    axis=1)
            for a, b in zip(bounds, bounds[1:])
        ]
        idx = jnp.minimum(jnp.minimum(pc[0], pc[1]), jnp.minimum(pc[2], pc[3]))
        vals.append(m)
        idxs.append(idx)
        x = jnp.where(iota == idx[:, None], _NEG_INF, x)
    topv_ref[0] = jnp.stack(vals, axis=1)
    topi_ref[0] = jnp.stack(idxs, axis=1)


def _merge_body(topv_ref, topi_ref, scores_ref, sc_ref, ix_ref, bm_ref, *, vocab):
    tv = topv_ref[...]
    ti = topi_ref[...]
    s = scores_ref[...]
    bsz, _, n_top = tv.shape
    prev = []
    cols = {}
    for g in range(_G):
        b0, b1 = g, g + _G
        cv = jnp.concatenate(
            [tv[:, b0, :] + s[:, b0][:, None], tv[:, b1, :] + s[:, b1][:, None]],
            axis=1)
        ci = jnp.concatenate([ti[:, b0, :], ti[:, b1, :]], axis=1)
        cflat = jnp.concatenate([ti[:, b0, :], ti[:, b1, :] + vocab], axis=1)
        cbeam = jnp.concatenate(
            [jnp.full((bsz, n_top), b0, jnp.int32),
             jnp.full((bsz, n_top), b1, jnp.int32)], axis=1)
        pen = jnp.zeros(cv.shape, jnp.float32)
        for p in prev:
            pen = pen + jnp.where(ci == p[:, None], 1.0, 0.0)
        adj = cv + _DIV * pen
        picked = []
        for r in range(2):
            m1 = jnp.max(adj, axis=1)
            sel = adj == m1[:, None]
            f1 = jnp.min(jnp.where(sel, cflat, _BIG), axis=1)
            ch = cflat == f1[:, None]
            isel = jnp.sum(jnp.where(ch, ci, 0), axis=1)
            bsel = jnp.sum(jnp.where(ch, cbeam, 0), axis=1)
            cols[r * _G + g] = (m1, isel, bsel)
            adj = jnp.where(ch, _NEG_INF, adj)
            picked.append(isel)
        prev.extend(picked)
    sc_ref[...] = jnp.stack([cols[c][0] for c in range(2 * _G)], axis=1)
    ix_ref[...] = jnp.stack([cols[c][1] for c in range(2 * _G)], axis=1)
    bm_ref[...] = jnp.stack([cols[c][2] for c in range(2 * _G)], axis=1)


def kernel(step, lprobs, scores, mask):
    bsz, nbeams, vocab = lprobs.shape
    scores2 = jnp.squeeze(scores, -1).astype(jnp.float32)
    mask_f = mask.astype(jnp.float32).reshape(bsz, 1, nbeams)

    topv, topi = pl.pallas_call(
        functools.partial(_topk_body, n_top=_NTOP),
        grid=(bsz,),
        in_specs=[
            pl.BlockSpec((1, nbeams, vocab), lambda b: (b, 0, 0)),
            pl.BlockSpec((1, 1, nbeams), lambda b: (b, 0, 0)),
        ],
        out_specs=[
            pl.BlockSpec((1, nbeams, _NTOP), lambda b: (b, 0, 0)),
            pl.BlockSpec((1, nbeams, _NTOP), lambda b: (b, 0, 0)),
        ],
        out_shape=[
            jax.ShapeDtypeStruct((bsz, nbeams, _NTOP), jnp.float32),
            jax.ShapeDtypeStruct((bsz, nbeams, _NTOP), jnp.int32),
        ],
    )(lprobs, mask_f)

    sc_out, ix_out, bm_out = pl.pallas_call(
        functools.partial(_merge_body, vocab=vocab),
        out_shape=[
            jax.ShapeDtypeStruct((bsz, 2 * _G), jnp.float32),
            jax.ShapeDtypeStruct((bsz, 2 * _G), jnp.int32),
            jax.ShapeDtypeStruct((bsz, 2 * _G), jnp.int32),
        ],
    )(topv, topi, scores2)
    return (sc_out, ix_out, bm_out)
